# baseline (speedup 1.0000x reference)
import functools

import jax
import jax.numpy as jnp
from jax import lax
from jax.experimental import pallas as pl
from jax.experimental.pallas import tpu as pltpu
from jax.experimental.pallas import tpu_sc as plsc

NUM_NODES = 1000000
H_DIM = 32
BATCH = 16384

_NC = 2
_NS = 16
_NW = _NC * _NS
_L = 16
_TW = 128
_PAD = 1000064
_RANGE = 32768
_BW = 512
_NBLK = _RANGE // _BW
_KCAP = 1024
_NIDX = BATCH // _L
_CLAMP = _PAD - _BW

_mesh = plsc.VectorSubcoreMesh(
    core_axis_name="c", subcore_axis_name="s", num_cores=_NC, num_subcores=_NS
)


@functools.partial(
    pl.kernel,
    mesh=_mesh,
    out_type=(
        jax.ShapeDtypeStruct((_NW * _KCAP * H_DIM,), jnp.float32),
        jax.ShapeDtypeStruct((_NW * _KCAP,), jnp.int32),
    ),
    scratch_types=[
        pltpu.VMEM((BATCH,), jnp.int32),
        pltpu.VMEM((_KCAP + 4 * _L,), jnp.int32),
        pltpu.VMEM((_KCAP,), jnp.int32),
        pltpu.VMEM((_KCAP + _L,), jnp.int32),
        pltpu.VMEM((_KCAP + _L,), jnp.int32),
        pltpu.VMEM((4, H_DIM, _BW), jnp.float32),
        pltpu.VMEM((_KCAP * H_DIM,), jnp.float32),
        pltpu.SemaphoreType.DMA,
        pltpu.SemaphoreType.DMA,
    ],
    compiler_params=pltpu.CompilerParams(
        use_tc_tiling_on_sc=True, needs_layout_passes=False
    ),
)
def _gather_kernel(
    idx_hbm, table_hbm, rows_hbm, pos_hbm,
    idx_v, acc_s, acc_p, mb_s, mb_o, bufs, rows_v, sem, osem,
):
    wid = lax.axis_index("s") * _NC + lax.axis_index("c")
    w0 = wid * _RANGE

    def blk_start(k):
        return jnp.minimum(w0 + k * _BW, jnp.int32(_CLAMP))

    def fetch(k, buf):
        @pl.when(w0 + k * _BW < jnp.int32(NUM_NODES))
        def _():
            start = pl.multiple_of(blk_start(k), _TW)
            for tr in range(H_DIM // 8):
                pltpu.async_copy(
                    table_hbm.at[pl.ds(tr * 8, 8), pl.ds(start, _BW)],
                    buf.at[pl.ds(tr * 8, 8)],
                    sem,
                )

    def drain(k, buf):
        @pl.when(w0 + k * _BW < jnp.int32(NUM_NODES))
        def _():
            for tr in range(H_DIM // 8):
                pltpu.make_async_copy(
                    table_hbm.at[pl.ds(0, 8), pl.ds(0, _BW)],
                    buf.at[pl.ds(tr * 8, 8)],
                    sem,
                ).wait()

    pltpu.sync_copy(idx_hbm, idx_v)
    for b in range(4):
        fetch(jnp.int32(b), bufs.at[b])

    dump = jnp.int32(BATCH + wid)
    for q in range(_KCAP // _L):
        acc_p[pl.ds(q * _L, _L)] = jnp.full((_L,), dump, jnp.int32)

    def filt(i, kacc):
        for u in range(4):
            q = i * 4 + u
            sv = idx_v[pl.ds(q * _L, _L)]
            m = (sv >> jnp.int32(15)) == wid
            bv = lax.iota(jnp.int32, _L) + q * _L
            plsc.store_compressed(acc_s.at[pl.ds(kacc, _L)], sv, mask=m)
            plsc.store_compressed(acc_p.at[pl.ds(kacc, _L)], bv, mask=m)
            kacc = kacc + plsc.all_reduce_population_count(m)[0]
        return kacc

    kacc = lax.fori_loop(0, _NIDX // 4, filt, jnp.int32(0))
    for u in range(4):
        acc_s[pl.ds(kacc + u * _L, _L)] = jnp.full((_L,), -1, jnp.int32)
    nacc4 = (kacc + jnp.int32(4 * _L - 1)) >> jnp.int32(6)

    def extract(k, buf):
        @pl.when(w0 + k * _BW < jnp.int32(NUM_NODES))
        def _():
            _extract_body(k, buf)

    def _extract_body(k, buf):
        start = blk_start(k)

        def match(i, mcnt):
            for u in range(4):
                q = i * 4 + u
                sv = acc_s[pl.ds(q * _L, _L)]
                ov = lax.iota(jnp.int32, _L) + q * _L
                m = (sv >= start) & (sv < start + jnp.int32(_BW))
                plsc.store_compressed(mb_s.at[pl.ds(mcnt, _L)], sv, mask=m)
                plsc.store_compressed(mb_o.at[pl.ds(mcnt, _L)], ov, mask=m)
                mcnt = mcnt + plsc.all_reduce_population_count(m)[0]
            return mcnt

        mcnt = lax.fori_loop(0, nacc4, match, jnp.int32(0))
        mb_s[pl.ds(mcnt, _L)] = jnp.full((_L,), 0, jnp.int32) + start
        mb_o[pl.ds(mcnt, _L)] = jnp.full((_L,), _KCAP - 1, jnp.int32)

        def pull(g, _):
            sv = mb_s[pl.ds(g * _L, _L)]
            ov = mb_o[pl.ds(g * _L, _L)]
            lvec = sv - start
            fv = ov * jnp.int32(H_DIM)
            for j in range(H_DIM):
                jvec = jnp.full((_L,), j, jnp.int32)
                vals = plsc.load_gather(buf, [jvec, lvec])
                plsc.store_scatter(rows_v, [fv + jnp.int32(j)], vals)
            return _

        ng = (mcnt + jnp.int32(_L - 1)) >> jnp.int32(4)
        lax.fori_loop(0, ng, pull, jnp.int32(0))

    def quad(t, _):
        for b in range(4):
            k = 4 * t + b
            drain(k, bufs.at[b])
            extract(k, bufs.at[b])

            @pl.when(t < _NBLK // 4 - 1)
            def _f():
                fetch(k + 4, bufs.at[b])

        return _

    lax.fori_loop(0, _NBLK // 4, quad, jnp.int32(0))

    rslice = pl.ds(wid * _KCAP * H_DIM, _KCAP * H_DIM)
    pslice = pl.ds(wid * _KCAP, _KCAP)
    pltpu.async_copy(rows_v, rows_hbm.at[rslice], osem)
    pltpu.async_copy(acc_p, pos_hbm.at[pslice], osem)
    pltpu.make_async_copy(rows_v, rows_hbm.at[rslice], osem).wait()
    pltpu.make_async_copy(acc_p, pos_hbm.at[pslice], osem).wait()


@functools.partial(
    pl.kernel,
    mesh=_mesh,
    out_type=jax.ShapeDtypeStruct((BATCH + _NW, H_DIM), jnp.float32),
    scratch_types=[
        pltpu.VMEM((_KCAP, H_DIM), jnp.float32),
        pltpu.VMEM((_KCAP,), jnp.int32),
        pltpu.SemaphoreType.DMA,
    ],
    compiler_params=pltpu.CompilerParams(
        use_tc_tiling_on_sc=False, needs_layout_passes=False
    ),
)
def _scatter_kernel(rows_hbm, pos_hbm, out_hbm, rows_v, pos_v, sem):
    wid = lax.axis_index("s") * _NC + lax.axis_index("c")
    base = wid * _KCAP
    pltpu.sync_copy(rows_hbm.at[pl.ds(base, _KCAP)], rows_v)
    pltpu.sync_copy(pos_hbm.at[pl.ds(base, _KCAP)], pos_v)
    pltpu.async_copy(rows_v, out_hbm.at[pos_v], sem)
    pltpu.make_async_copy(rows_v, out_hbm.at[pos_v], sem).wait()


def kernel(g, h, r, norm, embedding):
    idx = jnp.squeeze(h).astype(jnp.int32)
    rows, pos = _gather_kernel(idx, embedding.T)
    out_pad = _scatter_kernel(rows.reshape(_NW * _KCAP, H_DIM), pos)
    return out_pad[:BATCH]


# speedup vs baseline: 1.0625x; 1.0625x over previous
import functools

import jax
import jax.numpy as jnp
from jax import lax
from jax.experimental import pallas as pl
from jax.experimental.pallas import tpu as pltpu
from jax.experimental.pallas import tpu_sc as plsc

NUM_NODES = 1000000
H_DIM = 32
BATCH = 16384

_NC = 2
_NS = 16
_NW = _NC * _NS
_BPW = BATCH // _NW
_C = 16
_NCHUNK = _BPW // _C
_L = 16
_TW = 128

_mesh = plsc.VectorSubcoreMesh(
    core_axis_name="c", subcore_axis_name="s", num_cores=_NC, num_subcores=_NS
)


@functools.partial(
    pl.kernel,
    mesh=_mesh,
    out_type=jax.ShapeDtypeStruct((H_DIM, BATCH), jnp.float32),
    scratch_types=[
        pltpu.VMEM((_BPW,), jnp.int32),
        pltpu.VMEM((_C, H_DIM, _TW), jnp.float32),
        pltpu.VMEM((H_DIM, _BPW), jnp.float32),
        pltpu.SemaphoreType.DMA,
    ],
    compiler_params=pltpu.CompilerParams(
        use_tc_tiling_on_sc=True, needs_layout_passes=False
    ),
)
def _gather_kernel(idx_hbm, table_hbm, out_hbm, idx_v, blocks_v, cols_v, sem):
    wid = lax.axis_index("s") * _NC + lax.axis_index("c")
    base = wid * _BPW
    pltpu.sync_copy(idx_hbm.at[pl.ds(base, _BPW)], idx_v)

    def chunk_body(c):
        cbase = c * _C
        for g in range(_C // _L):
            starts = idx_v[pl.ds(cbase + g * _L, _L)] & jnp.int32(-_TW)
            for i in range(_L):
                start = pl.multiple_of(starts[i], _TW)
                pltpu.async_copy(
                    table_hbm.at[:, pl.ds(start, _TW)],
                    blocks_v.at[g * _L + i],
                    sem,
                )
        for i in range(_C):
            pltpu.make_async_copy(
                table_hbm.at[:, pl.ds(0, _TW)], blocks_v.at[i], sem
            ).wait()
        for g in range(_C // _L):
            lvec = idx_v[pl.ds(cbase + g * _L, _L)] & jnp.int32(_TW - 1)
            bvec = lax.iota(jnp.int32, _L) + jnp.int32(g * _L)
            for j in range(H_DIM):
                jvec = jnp.full((_L,), j, jnp.int32)
                vals = plsc.load_gather(blocks_v, [bvec, jvec, lvec])
                cols_v[j, pl.ds(cbase + g * _L, _L)] = vals

    pl.loop(0, _NCHUNK)(chunk_body)
    pltpu.sync_copy(cols_v, out_hbm.at[:, pl.ds(base, _BPW)])


def kernel(g, h, r, norm, embedding):
    idx = jnp.squeeze(h).astype(jnp.int32)
    out_t = _gather_kernel(idx, embedding.T)
    return out_t.T
